# baseline (speedup 1.0000x reference)
import functools

import jax
import jax.numpy as jnp
from jax import lax
from jax.experimental import pallas as pl
from jax.experimental.pallas import tpu as pltpu
from jax.experimental.pallas import tpu_sc as plsc

N = 10000
E = 320000
D = 128
R = 200
B = 16384

NC = 2
NS = 16
NW = NC * NS
CH = 128
NCH = 80
EPW = NCH * CH
E_PAD = EPW * NW
N_PAD = 10240
NPS = N_PAD // NS
ZR = 64
BPW = B // NW
GCH = 128
NGCH = BPW // GCH

_mesh = lambda: plsc.VectorSubcoreMesh(
    core_axis_name="c", subcore_axis_name="s", num_cores=NC, num_subcores=NS
)


def _wid():
    return lax.axis_index("c") * NS + lax.axis_index("s")


@functools.partial(
    pl.kernel,
    out_type=jax.ShapeDtypeStruct((NC, N_PAD, D), jnp.float32),
    mesh=_mesh(),
    scratch_types=[
        pltpu.VMEM((2, CH), jnp.int32),
        pltpu.VMEM((CH, D), jnp.float32),
        pltpu.VMEM((ZR, D), jnp.float32),
        pltpu.VMEM_SHARED((N_PAD, D), jnp.float32),
        pltpu.SemaphoreType.DMA,
        pltpu.SemaphoreType.DMA,
    ],
)
def _sc_deg(dst_hbm, ones_hbm, out_hbm, dst_v, ones_v, zbuf, accum, semi0, semi1):
    cid = lax.axis_index("c")
    sid = lax.axis_index("s")
    wid = cid * NS + sid
    semi = (semi0, semi1)

    def fill(i, carry):
        for j in range(D // 16):
            zbuf[i, pl.ds(j * 16, 16)] = jnp.zeros((16,), jnp.float32)
        return carry

    lax.fori_loop(0, ZR, fill, 0)
    pltpu.sync_copy(ones_hbm, ones_v)

    def zero(j, carry):
        pltpu.sync_copy(zbuf, accum.at[pl.ds(sid * NPS + j * ZR, ZR)])
        return carry

    lax.fori_loop(0, NPS // ZR, zero, 0)
    plsc.subcore_barrier()

    def start_idx(c, p):
        base = pl.multiple_of(wid * EPW + c * CH, 8)
        pltpu.async_copy(dst_hbm.at[pl.ds(base, CH)], dst_v.at[p], semi[p])

    def wait_idx(p):
        pltpu.make_async_copy(dst_hbm.at[pl.ds(0, CH)], dst_v.at[p], semi[p]).wait()

    def chunk(c, p):
        wait_idx(p)

        @pl.when(c + 1 < NCH)
        def _():
            start_idx(c + 1, 1 - p)

        pltpu.sync_copy(ones_v, accum.at[dst_v.at[p]], add=True)

    start_idx(0, 0)

    def body(j, carry):
        chunk(2 * j, 0)
        chunk(2 * j + 1, 1)
        return carry

    lax.fori_loop(0, NCH // 2, body, 0)
    plsc.subcore_barrier()
    pltpu.sync_copy(
        accum.at[pl.ds(sid * NPS, NPS)],
        out_hbm.at[cid, pl.ds(sid * NPS, NPS)],
    )


@functools.partial(
    pl.kernel,
    out_type=jax.ShapeDtypeStruct((NC, N_PAD, D), jnp.float32),
    mesh=_mesh(),
    scratch_types=[
        pltpu.VMEM((2, CH), jnp.int32),
        pltpu.VMEM((2, CH), jnp.int32),
        pltpu.VMEM((2, CH, D), jnp.float32),
        pltpu.VMEM((ZR, D), jnp.float32),
        pltpu.VMEM_SHARED((N_PAD, D), jnp.float32),
        pltpu.SemaphoreType.DMA,
        pltpu.SemaphoreType.DMA,
        pltpu.SemaphoreType.DMA,
        pltpu.SemaphoreType.DMA,
    ],
)
def _sc_msg(src_hbm, dst_hbm, hp_hbm, out_hbm, src_v, dst_v, rows_v, zbuf,
            accum, semi0, semi1, semg0, semg1):
    cid = lax.axis_index("c")
    sid = lax.axis_index("s")
    wid = cid * NS + sid
    semi = (semi0, semi1)
    semg = (semg0, semg1)

    def fill(i, carry):
        for j in range(D // 16):
            zbuf[i, pl.ds(j * 16, 16)] = jnp.zeros((16,), jnp.float32)
        return carry

    lax.fori_loop(0, ZR, fill, 0)

    def zero(j, carry):
        pltpu.sync_copy(zbuf, accum.at[pl.ds(sid * NPS + j * ZR, ZR)])
        return carry

    lax.fori_loop(0, NPS // ZR, zero, 0)
    plsc.subcore_barrier()

    def start_idx(c, p):
        base = pl.multiple_of(wid * EPW + c * CH, 8)
        pltpu.async_copy(src_hbm.at[pl.ds(base, CH)], src_v.at[p], semi[p])
        pltpu.async_copy(dst_hbm.at[pl.ds(base, CH)], dst_v.at[p], semi[p])

    def wait_idx(p):
        pltpu.make_async_copy(src_hbm.at[pl.ds(0, CH)], src_v.at[p], semi[p]).wait()
        pltpu.make_async_copy(dst_hbm.at[pl.ds(0, CH)], dst_v.at[p], semi[p]).wait()

    def start_gather(p):
        pltpu.async_copy(hp_hbm.at[src_v.at[p]], rows_v.at[p], semg[p])

    def wait_gather(p):
        pltpu.make_async_copy(hp_hbm.at[pl.ds(0, CH)], rows_v.at[p], semg[p]).wait()

    start_idx(0, 0)
    wait_idx(0)
    start_gather(0)
    start_idx(1, 1)

    def chunk(c, p):
        @pl.when(c + 1 < NCH)
        def _():
            wait_idx(1 - p)
            start_gather(1 - p)

        wait_gather(p)
        pltpu.sync_copy(rows_v.at[p], accum.at[dst_v.at[p]], add=True)

        @pl.when(c + 2 < NCH)
        def _():
            start_idx(c + 2, p)

    def body(j, carry):
        chunk(2 * j, 0)
        chunk(2 * j + 1, 1)
        return carry

    lax.fori_loop(0, NCH // 2, body, 0)
    plsc.subcore_barrier()
    pltpu.sync_copy(
        accum.at[pl.ds(sid * NPS, NPS)],
        out_hbm.at[cid, pl.ds(sid * NPS, NPS)],
    )


@functools.partial(
    pl.kernel,
    out_type=(
        jax.ShapeDtypeStruct((B, D), jnp.float32),
        jax.ShapeDtypeStruct((B, D), jnp.float32),
    ),
    mesh=_mesh(),
    scratch_types=[
        pltpu.VMEM((GCH,), jnp.int32),
        pltpu.VMEM((GCH,), jnp.int32),
        pltpu.VMEM((GCH, D), jnp.float32),
        pltpu.SemaphoreType.DMA,
    ],
)
def _sc_gather(z_hbm, head_hbm, tail_hbm, pos_hbm, zhs_hbm, zt_hbm,
               idx_v, pos_v, rows_v, sem):
    wid = _wid()
    for c in range(NGCH):
        base = pl.multiple_of(wid * BPW + c * GCH, 8)
        pltpu.sync_copy(head_hbm.at[pl.ds(base, GCH)], idx_v)
        pltpu.sync_copy(pos_hbm.at[pl.ds(base, GCH)], pos_v)
        pltpu.async_copy(z_hbm.at[idx_v], rows_v, sem).wait()
        pltpu.sync_copy(rows_v, zhs_hbm.at[pos_v])
    for c in range(NGCH):
        base = pl.multiple_of(wid * BPW + c * GCH, 8)
        pltpu.sync_copy(tail_hbm.at[pl.ds(base, GCH)], idx_v)
        pltpu.async_copy(z_hbm.at[idx_v], rows_v, sem).wait()
        pltpu.sync_copy(rows_v, zt_hbm.at[pl.ds(base, GCH)])


@functools.partial(
    pl.kernel,
    out_type=jax.ShapeDtypeStruct((B, D), jnp.float32),
    mesh=_mesh(),
    scratch_types=[
        pltpu.VMEM((GCH,), jnp.int32),
        pltpu.VMEM((GCH, D), jnp.float32),
        pltpu.SemaphoreType.DMA,
    ],
)
def _sc_unsort(tmps_hbm, pos_hbm, out_hbm, pos_v, rows_v, sem):
    wid = _wid()
    for c in range(NGCH):
        base = pl.multiple_of(wid * BPW + c * GCH, 8)
        pltpu.sync_copy(pos_hbm.at[pl.ds(base, GCH)], pos_v)
        pltpu.async_copy(tmps_hbm.at[pos_v], rows_v, sem).wait()
        pltpu.sync_copy(rows_v, out_hbm.at[pl.ds(base, GCH)])


def _dinv_of(degp_ref):
    deg = (degp_ref[0] + degp_ref[1])[:N, 0:1]
    return jnp.where(deg > 0, lax.rsqrt(deg), 0.0)


def _tc_pre_body(x0_ref, w1_ref, degp_ref, out_ref):
    dinv = _dinv_of(degp_ref)
    h = jnp.dot(x0_ref[...], w1_ref[...], preferred_element_type=jnp.float32)
    out_ref[...] = h * dinv


def _tc_mid_body(s1p_ref, degp_ref, b1_ref, w2_ref, out_ref):
    dinv = _dinv_of(degp_ref)
    s1 = (s1p_ref[0] + s1p_ref[1])[:N]
    z1 = jnp.maximum(s1 * dinv + b1_ref[...], 0.0)
    h2 = jnp.dot(z1, w2_ref[...], preferred_element_type=jnp.float32)
    out_ref[...] = h2 * dinv


def _tc_fin_body(s2p_ref, degp_ref, b2_ref, relw_ref, out_ref, relw_out):
    dinv = _dinv_of(degp_ref)
    out_ref[...] = (s2p_ref[0] + s2p_ref[1])[:N] * dinv + b2_ref[...]
    relw_out[...] = relw_ref[...].astype(jnp.bfloat16)


BT = 256
T = B // BT


def _tc_sorta_body(rel_ref, pn_out, total_out, acc_ref):
    t = pl.program_id(0)

    @pl.when(t == 0)
    def _():
        acc_ref[...] = jnp.zeros_like(acc_ref)

    rel_t = rel_ref[...]
    iota_r = lax.broadcasted_iota(jnp.int32, (1, R), 1)
    ohf = (rel_t == iota_r).astype(jnp.float32)
    ohb = ohf.astype(jnp.bfloat16)
    ii = lax.broadcasted_iota(jnp.int32, (BT, BT), 0)
    jj = lax.broadcasted_iota(jnp.int32, (BT, BT), 1)
    lstrict = (jj < ii).astype(jnp.bfloat16)
    prefix = jnp.dot(lstrict, ohb, preferred_element_type=jnp.float32)
    rank = jnp.sum(prefix * ohf, axis=1, keepdims=True)
    base = jnp.sum(acc_ref[...] * ohf, axis=1, keepdims=True)
    pn_out[...] = base + rank
    acc_ref[...] += jnp.sum(ohf, axis=0, keepdims=True)
    total_out[...] = acc_ref[...]


def _tc_sortb_body(pn_ref, total_ref, rel_ref, pos_out, cumend_out, bounds_out):
    total = total_ref[...]
    ur = (lax.broadcasted_iota(jnp.int32, (R, R), 0)
          <= lax.broadcasted_iota(jnp.int32, (R, R), 1)).astype(jnp.float32)
    cumend = jnp.dot(total, ur, preferred_element_type=jnp.float32)
    goff = cumend - total
    iota_r = lax.broadcasted_iota(jnp.int32, (1, R), 1)
    ohf = (rel_ref[...] == iota_r).astype(jnp.float32)
    posadd = jnp.sum(ohf * goff, axis=1, keepdims=True)
    pos_out[...] = (pn_ref[...] + posadd).astype(jnp.int32)
    cumend_out[...] = cumend
    starts = lax.broadcasted_iota(jnp.int32, (T, 1), 0).astype(jnp.float32) * BT
    rlo = jnp.sum((cumend <= starts).astype(jnp.float32), axis=1, keepdims=True)
    rhi = jnp.sum((cumend <= starts + (BT - 1)).astype(jnp.float32),
                  axis=1, keepdims=True)
    bounds_out[...] = jnp.concatenate([rlo, rhi], axis=1).astype(jnp.int32)


def _tc_score_body(bounds_ref, cumend_ref, zhs_ref, w_ref, out_ref):
    t = pl.program_id(0)
    rlo = bounds_ref[t, 0]
    rhi = bounds_ref[t, 1]
    slot = (lax.broadcasted_iota(jnp.int32, (BT, 1), 0).astype(jnp.float32)
            + jnp.float32(BT) * t.astype(jnp.float32))
    relslot = jnp.sum((cumend_ref[...] <= slot).astype(jnp.float32),
                      axis=1, keepdims=True)
    out_ref[...] = jnp.zeros_like(out_ref)
    zh_bf = zhs_ref[...].astype(jnp.bfloat16)

    def rbody(r, carry):
        m = relslot == r.astype(jnp.float32)
        zm = jnp.where(m, zh_bf, jnp.bfloat16(0))
        out_ref[...] += jnp.dot(zm, w_ref[r], preferred_element_type=jnp.float32)
        return carry

    lax.fori_loop(rlo, rhi + 1, rbody, 0)


def _tc_final_body(tmp_ref, zt_ref, out_ref):
    out_ref[...] = jnp.sum(tmp_ref[...] * zt_ref[...], axis=1, keepdims=True)


def kernel(edge_index, rel_ids, head, tail, x0, W1, b1, W2, b2, rel_W):
    src = jnp.concatenate([edge_index[0], jnp.zeros((E_PAD - E,), jnp.int32)])
    dst = jnp.concatenate([edge_index[1], jnp.full((E_PAD - E,), N, jnp.int32)])
    degp = _sc_deg(dst, jnp.ones((CH, D), jnp.float32))

    h1p = pl.pallas_call(
        _tc_pre_body,
        out_shape=jax.ShapeDtypeStruct((N, D), jnp.float32),
    )(x0, W1, degp)

    s1p = _sc_msg(src, dst, h1p)

    h2p = pl.pallas_call(
        _tc_mid_body,
        out_shape=jax.ShapeDtypeStruct((N, D), jnp.float32),
    )(s1p, degp, b1, W2)

    s2p = _sc_msg(src, dst, h2p)

    z2, relw_bf = pl.pallas_call(
        _tc_fin_body,
        out_shape=(
            jax.ShapeDtypeStruct((N, D), jnp.float32),
            jax.ShapeDtypeStruct((R, D, D), jnp.bfloat16),
        ),
    )(s2p, degp, b2, rel_W)

    rel2 = rel_ids.reshape(B, 1)

    pn, total = pl.pallas_call(
        _tc_sorta_body,
        grid=(T,),
        in_specs=[pl.BlockSpec((BT, 1), lambda t: (t, 0))],
        out_specs=(
            pl.BlockSpec((BT, 1), lambda t: (t, 0)),
            pl.BlockSpec((1, R), lambda t: (0, 0)),
        ),
        out_shape=(
            jax.ShapeDtypeStruct((B, 1), jnp.float32),
            jax.ShapeDtypeStruct((1, R), jnp.float32),
        ),
        scratch_shapes=[pltpu.VMEM((1, R), jnp.float32)],
    )(rel2)

    pos, cumend, bounds = pl.pallas_call(
        _tc_sortb_body,
        out_shape=(
            jax.ShapeDtypeStruct((B, 1), jnp.int32),
            jax.ShapeDtypeStruct((1, R), jnp.float32),
            jax.ShapeDtypeStruct((T, 2), jnp.int32),
        ),
    )(pn, total, rel2)

    zhs, zt = _sc_gather(z2, head, tail, pos.reshape(B))

    tmps = pl.pallas_call(
        _tc_score_body,
        grid=(T,),
        in_specs=[
            pl.BlockSpec(memory_space=pltpu.SMEM),
            pl.BlockSpec((1, R), lambda t: (0, 0)),
            pl.BlockSpec((BT, D), lambda t: (t, 0)),
            pl.BlockSpec((R, D, D), lambda t: (0, 0, 0)),
        ],
        out_specs=pl.BlockSpec((BT, D), lambda t: (t, 0)),
        out_shape=jax.ShapeDtypeStruct((B, D), jnp.float32),
    )(bounds, cumend, zhs, relw_bf)

    tmp = _sc_unsort(tmps, pos.reshape(B))

    scores = pl.pallas_call(
        _tc_final_body,
        out_shape=jax.ShapeDtypeStruct((B, 1), jnp.float32),
    )(tmp, zt)

    return scores.reshape(B)


# speedup vs baseline: 1.0008x; 1.0008x over previous
import functools

import jax
import jax.numpy as jnp
from jax import lax
from jax.experimental import pallas as pl
from jax.experimental.pallas import tpu as pltpu
from jax.experimental.pallas import tpu_sc as plsc

N = 10000
E = 320000
D = 128
R = 200
B = 16384

NC = 2
NS = 16
NW = NC * NS
CH = 128
NCH = 80
EPW = NCH * CH
E_PAD = EPW * NW
N_PAD = 10240
NPS = N_PAD // NS
ZR = 64
BPW = B // NW
GCH = 128
NGCH = BPW // GCH

_mesh = lambda: plsc.VectorSubcoreMesh(
    core_axis_name="c", subcore_axis_name="s", num_cores=NC, num_subcores=NS
)


def _wid():
    return lax.axis_index("c") * NS + lax.axis_index("s")


@functools.partial(
    pl.kernel,
    out_type=jax.ShapeDtypeStruct((NC, N_PAD, D), jnp.float32),
    mesh=_mesh(),
    scratch_types=[
        pltpu.VMEM((2, CH), jnp.int32),
        pltpu.VMEM((CH, D), jnp.float32),
        pltpu.VMEM((ZR, D), jnp.float32),
        pltpu.VMEM_SHARED((N_PAD, D), jnp.float32),
        pltpu.SemaphoreType.DMA,
        pltpu.SemaphoreType.DMA,
    ],
)
def _sc_deg(dst_hbm, ones_hbm, out_hbm, dst_v, ones_v, zbuf, accum, semi0, semi1):
    cid = lax.axis_index("c")
    sid = lax.axis_index("s")
    wid = cid * NS + sid
    semi = (semi0, semi1)

    def fill(i, carry):
        for j in range(D // 16):
            zbuf[i, pl.ds(j * 16, 16)] = jnp.zeros((16,), jnp.float32)
        return carry

    lax.fori_loop(0, ZR, fill, 0)
    pltpu.sync_copy(ones_hbm, ones_v)

    def zero(j, carry):
        pltpu.sync_copy(zbuf, accum.at[pl.ds(sid * NPS + j * ZR, ZR)])
        return carry

    lax.fori_loop(0, NPS // ZR, zero, 0)
    plsc.subcore_barrier()

    def start_idx(c, p):
        base = pl.multiple_of(wid * EPW + c * CH, 8)
        pltpu.async_copy(dst_hbm.at[pl.ds(base, CH)], dst_v.at[p], semi[p])

    def wait_idx(p):
        pltpu.make_async_copy(dst_hbm.at[pl.ds(0, CH)], dst_v.at[p], semi[p]).wait()

    def chunk(c, p):
        wait_idx(p)

        @pl.when(c + 1 < NCH)
        def _():
            start_idx(c + 1, 1 - p)

        pltpu.sync_copy(ones_v, accum.at[dst_v.at[p]], add=True)

    start_idx(0, 0)

    def body(j, carry):
        chunk(2 * j, 0)
        chunk(2 * j + 1, 1)
        return carry

    lax.fori_loop(0, NCH // 2, body, 0)
    plsc.subcore_barrier()
    pltpu.sync_copy(
        accum.at[pl.ds(sid * NPS, NPS)],
        out_hbm.at[cid, pl.ds(sid * NPS, NPS)],
    )


@functools.partial(
    pl.kernel,
    out_type=jax.ShapeDtypeStruct((NC, N_PAD, D), jnp.float32),
    mesh=_mesh(),
    scratch_types=[
        pltpu.VMEM((2, CH), jnp.int32),
        pltpu.VMEM((2, CH), jnp.int32),
        pltpu.VMEM((2, CH, D), jnp.float32),
        pltpu.VMEM((ZR, D), jnp.float32),
        pltpu.VMEM_SHARED((N_PAD, D), jnp.float32),
        pltpu.SemaphoreType.DMA,
        pltpu.SemaphoreType.DMA,
        pltpu.SemaphoreType.DMA,
        pltpu.SemaphoreType.DMA,
    ],
)
def _sc_msg(src_hbm, dst_hbm, hp_hbm, out_hbm, src_v, dst_v, rows_v, zbuf,
            accum, semi0, semi1, semg0, semg1):
    cid = lax.axis_index("c")
    sid = lax.axis_index("s")
    wid = cid * NS + sid
    semi = (semi0, semi1)
    semg = (semg0, semg1)

    def fill(i, carry):
        for j in range(D // 16):
            zbuf[i, pl.ds(j * 16, 16)] = jnp.zeros((16,), jnp.float32)
        return carry

    lax.fori_loop(0, ZR, fill, 0)

    def zero(j, carry):
        pltpu.sync_copy(zbuf, accum.at[pl.ds(sid * NPS + j * ZR, ZR)])
        return carry

    lax.fori_loop(0, NPS // ZR, zero, 0)
    plsc.subcore_barrier()

    def start_idx(c, p):
        base = pl.multiple_of(wid * EPW + c * CH, 8)
        pltpu.async_copy(src_hbm.at[pl.ds(base, CH)], src_v.at[p], semi[p])
        pltpu.async_copy(dst_hbm.at[pl.ds(base, CH)], dst_v.at[p], semi[p])

    def wait_idx(p):
        pltpu.make_async_copy(src_hbm.at[pl.ds(0, CH)], src_v.at[p], semi[p]).wait()
        pltpu.make_async_copy(dst_hbm.at[pl.ds(0, CH)], dst_v.at[p], semi[p]).wait()

    def start_gather(p):
        pltpu.async_copy(hp_hbm.at[src_v.at[p]], rows_v.at[p], semg[p])

    def wait_gather(p):
        pltpu.make_async_copy(hp_hbm.at[pl.ds(0, CH)], rows_v.at[p], semg[p]).wait()

    start_idx(0, 0)
    wait_idx(0)
    start_gather(0)
    start_idx(1, 1)

    def chunk(c, p):
        @pl.when(c + 1 < NCH)
        def _():
            wait_idx(1 - p)
            start_gather(1 - p)

        wait_gather(p)
        pltpu.sync_copy(rows_v.at[p], accum.at[dst_v.at[p]], add=True)

        @pl.when(c + 2 < NCH)
        def _():
            start_idx(c + 2, p)

    def body(j, carry):
        chunk(2 * j, 0)
        chunk(2 * j + 1, 1)
        return carry

    lax.fori_loop(0, NCH // 2, body, 0)
    plsc.subcore_barrier()
    pltpu.sync_copy(
        accum.at[pl.ds(sid * NPS, NPS)],
        out_hbm.at[cid, pl.ds(sid * NPS, NPS)],
    )


@functools.partial(
    pl.kernel,
    out_type=(
        jax.ShapeDtypeStruct((B, D), jnp.float32),
        jax.ShapeDtypeStruct((B, D), jnp.float32),
    ),
    mesh=_mesh(),
    scratch_types=[
        pltpu.VMEM((GCH,), jnp.int32),
        pltpu.VMEM((GCH,), jnp.int32),
        pltpu.VMEM((GCH, D), jnp.float32),
        pltpu.SemaphoreType.DMA,
    ],
)
def _sc_gather(z_hbm, head_hbm, tail_hbm, pos_hbm, zhs_hbm, zt_hbm,
               idx_v, pos_v, rows_v, sem):
    wid = _wid()
    for c in range(NGCH):
        base = pl.multiple_of(wid * BPW + c * GCH, 8)
        pltpu.sync_copy(head_hbm.at[pl.ds(base, GCH)], idx_v)
        pltpu.sync_copy(pos_hbm.at[pl.ds(base, GCH)], pos_v)
        pltpu.async_copy(z_hbm.at[idx_v], rows_v, sem).wait()
        pltpu.sync_copy(rows_v, zhs_hbm.at[pos_v])
    for c in range(NGCH):
        base = pl.multiple_of(wid * BPW + c * GCH, 8)
        pltpu.sync_copy(tail_hbm.at[pl.ds(base, GCH)], idx_v)
        pltpu.async_copy(z_hbm.at[idx_v], rows_v, sem).wait()
        pltpu.sync_copy(rows_v, zt_hbm.at[pl.ds(base, GCH)])


@functools.partial(
    pl.kernel,
    out_type=jax.ShapeDtypeStruct((B, D), jnp.float32),
    mesh=_mesh(),
    scratch_types=[
        pltpu.VMEM((GCH,), jnp.int32),
        pltpu.VMEM((GCH, D), jnp.float32),
        pltpu.SemaphoreType.DMA,
    ],
)
def _sc_unsort(tmps_hbm, pos_hbm, out_hbm, pos_v, rows_v, sem):
    wid = _wid()
    for c in range(NGCH):
        base = pl.multiple_of(wid * BPW + c * GCH, 8)
        pltpu.sync_copy(pos_hbm.at[pl.ds(base, GCH)], pos_v)
        pltpu.async_copy(tmps_hbm.at[pos_v], rows_v, sem).wait()
        pltpu.sync_copy(rows_v, out_hbm.at[pl.ds(base, GCH)])


def _dinv_of(degp_ref):
    deg = (degp_ref[0] + degp_ref[1])[:N, 0:1]
    return jnp.where(deg > 0, lax.rsqrt(deg), 0.0)


def _tc_pre_body(x0_ref, w1_ref, degp_ref, out_ref):
    dinv = _dinv_of(degp_ref)
    h = jnp.dot(x0_ref[...], w1_ref[...], preferred_element_type=jnp.float32)
    out_ref[...] = h * dinv


def _tc_mid_body(s1p_ref, degp_ref, b1_ref, w2_ref, out_ref):
    dinv = _dinv_of(degp_ref)
    s1 = (s1p_ref[0] + s1p_ref[1])[:N]
    z1 = jnp.maximum(s1 * dinv + b1_ref[...], 0.0)
    h2 = jnp.dot(z1, w2_ref[...], preferred_element_type=jnp.float32)
    out_ref[...] = h2 * dinv


def _tc_fin_body(s2p_ref, degp_ref, b2_ref, relw_ref, out_ref, relw_out):
    dinv = _dinv_of(degp_ref)
    out_ref[...] = (s2p_ref[0] + s2p_ref[1])[:N] * dinv + b2_ref[...]
    relw_out[...] = relw_ref[...].astype(jnp.bfloat16)


BT = 256
T = B // BT


def _tc_sorta_body(rel_ref, pn_out, total_out, acc_ref):
    t = pl.program_id(0)

    @pl.when(t == 0)
    def _():
        acc_ref[...] = jnp.zeros_like(acc_ref)

    rel_t = rel_ref[...]
    iota_r = lax.broadcasted_iota(jnp.int32, (1, R), 1)
    ohf = (rel_t == iota_r).astype(jnp.float32)
    ohb = ohf.astype(jnp.bfloat16)
    ii = lax.broadcasted_iota(jnp.int32, (BT, BT), 0)
    jj = lax.broadcasted_iota(jnp.int32, (BT, BT), 1)
    lstrict = (jj < ii).astype(jnp.bfloat16)
    prefix = jnp.dot(lstrict, ohb, preferred_element_type=jnp.float32)
    rank = jnp.sum(prefix * ohf, axis=1, keepdims=True)
    base = jnp.sum(acc_ref[...] * ohf, axis=1, keepdims=True)
    pn_out[...] = base + rank
    acc_ref[...] += jnp.sum(ohf, axis=0, keepdims=True)
    total_out[...] = acc_ref[...]


def _tc_sortb_body(pn_ref, total_ref, rel_ref, pos_out, cumend_out, bounds_out):
    total = total_ref[...]
    ur = (lax.broadcasted_iota(jnp.int32, (R, R), 0)
          <= lax.broadcasted_iota(jnp.int32, (R, R), 1)).astype(jnp.float32)
    cumend = jnp.dot(total, ur, preferred_element_type=jnp.float32)
    goff = cumend - total
    iota_r = lax.broadcasted_iota(jnp.int32, (1, R), 1)
    ohf = (rel_ref[...] == iota_r).astype(jnp.float32)
    posadd = jnp.sum(ohf * goff, axis=1, keepdims=True)
    pos_out[...] = (pn_ref[...] + posadd).astype(jnp.int32)
    cumend_out[...] = cumend
    starts = lax.broadcasted_iota(jnp.int32, (T, 1), 0).astype(jnp.float32) * BT
    rlo = jnp.sum((cumend <= starts).astype(jnp.float32), axis=1, keepdims=True)
    rhi = jnp.sum((cumend <= starts + (BT - 1)).astype(jnp.float32),
                  axis=1, keepdims=True)
    bounds_out[...] = jnp.concatenate([rlo, rhi], axis=1).astype(jnp.int32)


def _tc_score_body(bounds_ref, cumend_ref, zhs_ref, w_ref, out_ref):
    t = pl.program_id(0)
    rlo = bounds_ref[t, 0]
    rhi = bounds_ref[t, 1]
    slot = (lax.broadcasted_iota(jnp.int32, (BT, 1), 0).astype(jnp.float32)
            + jnp.float32(BT) * t.astype(jnp.float32))
    relslot = jnp.sum((cumend_ref[...] <= slot).astype(jnp.float32),
                      axis=1, keepdims=True)
    out_ref[...] = jnp.zeros_like(out_ref)
    zh_bf = zhs_ref[...].astype(jnp.bfloat16)

    def rbody(r, carry):
        m = relslot == r.astype(jnp.float32)
        zm = jnp.where(m, zh_bf, jnp.bfloat16(0))
        out_ref[...] += jnp.dot(zm, w_ref[r], preferred_element_type=jnp.float32)
        return carry

    lax.fori_loop(rlo, rhi + 1, rbody, 0)


def _tc_final_body(tmp_ref, zt_ref, out_ref):
    out_ref[...] = jnp.sum(tmp_ref[...] * zt_ref[...], axis=1, keepdims=True)


def kernel(edge_index, rel_ids, head, tail, x0, W1, b1, W2, b2, rel_W):
    src = jnp.concatenate([edge_index[0], jnp.zeros((E_PAD - E,), jnp.int32)])
    pad_dst = N + jnp.arange(E_PAD - E, dtype=jnp.int32) % (N_PAD - N)
    dst = jnp.concatenate([edge_index[1], pad_dst])
    degp = _sc_deg(dst, jnp.ones((CH, D), jnp.float32))

    h1p = pl.pallas_call(
        _tc_pre_body,
        out_shape=jax.ShapeDtypeStruct((N, D), jnp.float32),
    )(x0, W1, degp)

    s1p = _sc_msg(src, dst, h1p)

    h2p = pl.pallas_call(
        _tc_mid_body,
        out_shape=jax.ShapeDtypeStruct((N, D), jnp.float32),
    )(s1p, degp, b1, W2)

    s2p = _sc_msg(src, dst, h2p)

    z2, relw_bf = pl.pallas_call(
        _tc_fin_body,
        out_shape=(
            jax.ShapeDtypeStruct((N, D), jnp.float32),
            jax.ShapeDtypeStruct((R, D, D), jnp.bfloat16),
        ),
    )(s2p, degp, b2, rel_W)

    rel2 = rel_ids.reshape(B, 1)

    pn, total = pl.pallas_call(
        _tc_sorta_body,
        grid=(T,),
        in_specs=[pl.BlockSpec((BT, 1), lambda t: (t, 0))],
        out_specs=(
            pl.BlockSpec((BT, 1), lambda t: (t, 0)),
            pl.BlockSpec((1, R), lambda t: (0, 0)),
        ),
        out_shape=(
            jax.ShapeDtypeStruct((B, 1), jnp.float32),
            jax.ShapeDtypeStruct((1, R), jnp.float32),
        ),
        scratch_shapes=[pltpu.VMEM((1, R), jnp.float32)],
    )(rel2)

    pos, cumend, bounds = pl.pallas_call(
        _tc_sortb_body,
        out_shape=(
            jax.ShapeDtypeStruct((B, 1), jnp.int32),
            jax.ShapeDtypeStruct((1, R), jnp.float32),
            jax.ShapeDtypeStruct((T, 2), jnp.int32),
        ),
    )(pn, total, rel2)

    zhs, zt = _sc_gather(z2, head, tail, pos.reshape(B))

    tmps = pl.pallas_call(
        _tc_score_body,
        grid=(T,),
        in_specs=[
            pl.BlockSpec(memory_space=pltpu.SMEM),
            pl.BlockSpec((1, R), lambda t: (0, 0)),
            pl.BlockSpec((BT, D), lambda t: (t, 0)),
            pl.BlockSpec((R, D, D), lambda t: (0, 0, 0)),
        ],
        out_specs=pl.BlockSpec((BT, D), lambda t: (t, 0)),
        out_shape=jax.ShapeDtypeStruct((B, D), jnp.float32),
    )(bounds, cumend, zhs, relw_bf)

    tmp = _sc_unsort(tmps, pos.reshape(B))

    scores = pl.pallas_call(
        _tc_final_body,
        out_shape=jax.ShapeDtypeStruct((B, 1), jnp.float32),
    )(tmp, zt)

    return scores.reshape(B)


# speedup vs baseline: 2.0377x; 2.0361x over previous
import functools

import jax
import jax.numpy as jnp
from jax import lax
from jax.experimental import pallas as pl
from jax.experimental.pallas import tpu as pltpu
from jax.experimental.pallas import tpu_sc as plsc

N = 10000
E = 320000
D = 128
R = 200
B = 16384

NC = 2
NS = 16
NW = NC * NS
CH = 80
NCH = 125
EPW = NCH * CH
E_PAD = EPW * NW
N_PAD = 10240
NPS = N_PAD // NS
ZR = 64
BPW = B // NW
GCH = 128
NGCH = BPW // GCH

_mesh = lambda: plsc.VectorSubcoreMesh(
    core_axis_name="c", subcore_axis_name="s", num_cores=NC, num_subcores=NS
)


def _wid():
    return lax.axis_index("c") * NS + lax.axis_index("s")


@functools.partial(
    pl.kernel,
    out_type=jax.ShapeDtypeStruct((NC, N_PAD, D), jnp.float32),
    mesh=_mesh(),
    scratch_types=[
        pltpu.VMEM((2, CH), jnp.int32),
        pltpu.VMEM((CH, D), jnp.float32),
        pltpu.VMEM((ZR, D), jnp.float32),
        pltpu.VMEM_SHARED((N_PAD, D), jnp.float32),
        pltpu.SemaphoreType.DMA,
        pltpu.SemaphoreType.DMA,
    ],
)
def _sc_deg(dst_hbm, ones_hbm, out_hbm, dst_v, ones_v, zbuf, accum, semi0, semi1):
    cid = lax.axis_index("c")
    sid = lax.axis_index("s")
    wid = cid * NS + sid
    semi = (semi0, semi1)

    def fill(i, carry):
        for j in range(D // 16):
            zbuf[i, pl.ds(j * 16, 16)] = jnp.zeros((16,), jnp.float32)
        return carry

    lax.fori_loop(0, ZR, fill, 0)
    pltpu.sync_copy(ones_hbm, ones_v)

    def zero(j, carry):
        pltpu.sync_copy(zbuf, accum.at[pl.ds(sid * NPS + j * ZR, ZR)])
        return carry

    lax.fori_loop(0, NPS // ZR, zero, 0)
    plsc.subcore_barrier()

    def start_idx(c, p):
        base = pl.multiple_of(wid * EPW + c * CH, 8)
        pltpu.async_copy(dst_hbm.at[pl.ds(base, CH)], dst_v.at[p], semi[p])

    def wait_idx(p):
        pltpu.make_async_copy(dst_hbm.at[pl.ds(0, CH)], dst_v.at[p], semi[p]).wait()

    def chunk(c, p):
        wait_idx(p)

        @pl.when(c + 1 < NCH)
        def _():
            start_idx(c + 1, 1 - p)

        pltpu.sync_copy(ones_v, accum.at[dst_v.at[p]], add=True)

    start_idx(0, 0)

    def body(j, carry):
        chunk(2 * j, 0)
        chunk(2 * j + 1, 1)
        return carry

    lax.fori_loop(0, NCH // 2, body, 0)
    wait_idx(0)
    pltpu.sync_copy(ones_v, accum.at[dst_v.at[0]], add=True)
    plsc.subcore_barrier()
    pltpu.sync_copy(
        accum.at[pl.ds(sid * NPS, NPS)],
        out_hbm.at[cid, pl.ds(sid * NPS, NPS)],
    )


@functools.partial(
    pl.kernel,
    out_type=jax.ShapeDtypeStruct((NC, N_PAD, D), jnp.float32),
    mesh=_mesh(),
    scratch_types=[
        pltpu.VMEM((2, CH), jnp.int32),
        pltpu.VMEM((2, CH), jnp.int32),
        pltpu.VMEM((2, CH, D), jnp.float32),
        pltpu.VMEM((ZR, D), jnp.float32),
        pltpu.VMEM_SHARED((N_PAD, D), jnp.float32),
        pltpu.SemaphoreType.DMA,
        pltpu.SemaphoreType.DMA,
        pltpu.SemaphoreType.DMA,
        pltpu.SemaphoreType.DMA,
    ],
)
def _sc_msg(src_hbm, dst_hbm, hp_hbm, out_hbm, src_v, dst_v, rows_v, zbuf,
            accum, semi0, semi1, semg0, semg1):
    cid = lax.axis_index("c")
    sid = lax.axis_index("s")
    wid = cid * NS + sid
    semi = (semi0, semi1)
    semg = (semg0, semg1)

    def fill(i, carry):
        for j in range(D // 16):
            zbuf[i, pl.ds(j * 16, 16)] = jnp.zeros((16,), jnp.float32)
        return carry

    lax.fori_loop(0, ZR, fill, 0)

    def zero(j, carry):
        pltpu.sync_copy(zbuf, accum.at[pl.ds(sid * NPS + j * ZR, ZR)])
        return carry

    lax.fori_loop(0, NPS // ZR, zero, 0)
    plsc.subcore_barrier()

    def start_idx(c, p):
        base = pl.multiple_of(wid * EPW + c * CH, 8)
        pltpu.async_copy(src_hbm.at[pl.ds(base, CH)], src_v.at[p], semi[p])
        pltpu.async_copy(dst_hbm.at[pl.ds(base, CH)], dst_v.at[p], semi[p])

    def wait_idx(p):
        pltpu.make_async_copy(src_hbm.at[pl.ds(0, CH)], src_v.at[p], semi[p]).wait()
        pltpu.make_async_copy(dst_hbm.at[pl.ds(0, CH)], dst_v.at[p], semi[p]).wait()

    def start_gather(p):
        pltpu.async_copy(hp_hbm.at[src_v.at[p]], rows_v.at[p], semg[p])

    def wait_gather(p):
        pltpu.make_async_copy(hp_hbm.at[pl.ds(0, CH)], rows_v.at[p], semg[p]).wait()

    start_idx(0, 0)
    wait_idx(0)
    start_gather(0)
    start_idx(1, 1)

    def chunk(c, p):
        @pl.when(c + 1 < NCH)
        def _():
            wait_idx(1 - p)
            start_gather(1 - p)

        wait_gather(p)
        pltpu.sync_copy(rows_v.at[p], accum.at[dst_v.at[p]], add=True)

        @pl.when(c + 2 < NCH)
        def _():
            start_idx(c + 2, p)

    def body(j, carry):
        chunk(2 * j, 0)
        chunk(2 * j + 1, 1)
        return carry

    lax.fori_loop(0, NCH // 2, body, 0)
    wait_gather(0)
    pltpu.sync_copy(rows_v.at[0], accum.at[dst_v.at[0]], add=True)
    plsc.subcore_barrier()
    pltpu.sync_copy(
        accum.at[pl.ds(sid * NPS, NPS)],
        out_hbm.at[cid, pl.ds(sid * NPS, NPS)],
    )


@functools.partial(
    pl.kernel,
    out_type=(
        jax.ShapeDtypeStruct((B, D), jnp.float32),
        jax.ShapeDtypeStruct((B, D), jnp.float32),
    ),
    mesh=_mesh(),
    scratch_types=[
        pltpu.VMEM((GCH,), jnp.int32),
        pltpu.VMEM((GCH,), jnp.int32),
        pltpu.VMEM((GCH, D), jnp.float32),
        pltpu.SemaphoreType.DMA,
    ],
)
def _sc_gather(z_hbm, head_hbm, tail_hbm, pos_hbm, zhs_hbm, zt_hbm,
               idx_v, pos_v, rows_v, sem):
    wid = _wid()
    for c in range(NGCH):
        base = pl.multiple_of(wid * BPW + c * GCH, 8)
        pltpu.sync_copy(head_hbm.at[pl.ds(base, GCH)], idx_v)
        pltpu.sync_copy(pos_hbm.at[pl.ds(base, GCH)], pos_v)
        pltpu.async_copy(z_hbm.at[idx_v], rows_v, sem).wait()
        pltpu.sync_copy(rows_v, zhs_hbm.at[pos_v])
    for c in range(NGCH):
        base = pl.multiple_of(wid * BPW + c * GCH, 8)
        pltpu.sync_copy(tail_hbm.at[pl.ds(base, GCH)], idx_v)
        pltpu.async_copy(z_hbm.at[idx_v], rows_v, sem).wait()
        pltpu.sync_copy(rows_v, zt_hbm.at[pl.ds(base, GCH)])


@functools.partial(
    pl.kernel,
    out_type=jax.ShapeDtypeStruct((B, D), jnp.float32),
    mesh=_mesh(),
    scratch_types=[
        pltpu.VMEM((GCH,), jnp.int32),
        pltpu.VMEM((GCH, D), jnp.float32),
        pltpu.SemaphoreType.DMA,
    ],
)
def _sc_unsort(tmps_hbm, pos_hbm, out_hbm, pos_v, rows_v, sem):
    wid = _wid()
    for c in range(NGCH):
        base = pl.multiple_of(wid * BPW + c * GCH, 8)
        pltpu.sync_copy(pos_hbm.at[pl.ds(base, GCH)], pos_v)
        pltpu.async_copy(tmps_hbm.at[pos_v], rows_v, sem).wait()
        pltpu.sync_copy(rows_v, out_hbm.at[pl.ds(base, GCH)])


def _dinv_of(degp_ref):
    deg = (degp_ref[0] + degp_ref[1])[:N, 0:1]
    return jnp.where(deg > 0, lax.rsqrt(deg), 0.0)


def _tc_pre_body(x0_ref, w1_ref, degp_ref, out_ref):
    dinv = _dinv_of(degp_ref)
    h = jnp.dot(x0_ref[...], w1_ref[...], preferred_element_type=jnp.float32)
    out_ref[...] = h * dinv


def _tc_mid_body(s1p_ref, degp_ref, b1_ref, w2_ref, out_ref):
    dinv = _dinv_of(degp_ref)
    s1 = (s1p_ref[0] + s1p_ref[1])[:N]
    z1 = jnp.maximum(s1 * dinv + b1_ref[...], 0.0)
    h2 = jnp.dot(z1, w2_ref[...], preferred_element_type=jnp.float32)
    out_ref[...] = h2 * dinv


def _tc_fin_body(s2p_ref, degp_ref, b2_ref, relw_ref, out_ref, relw_out):
    dinv = _dinv_of(degp_ref)
    out_ref[...] = (s2p_ref[0] + s2p_ref[1])[:N] * dinv + b2_ref[...]
    relw_out[...] = relw_ref[...].astype(jnp.bfloat16)


BT = 256
T = B // BT


def _tc_sorta_body(rel_ref, pn_out, total_out, acc_ref):
    t = pl.program_id(0)

    @pl.when(t == 0)
    def _():
        acc_ref[...] = jnp.zeros_like(acc_ref)

    rel_t = rel_ref[...]
    iota_r = lax.broadcasted_iota(jnp.int32, (1, R), 1)
    ohf = (rel_t == iota_r).astype(jnp.float32)
    ohb = ohf.astype(jnp.bfloat16)
    ii = lax.broadcasted_iota(jnp.int32, (BT, BT), 0)
    jj = lax.broadcasted_iota(jnp.int32, (BT, BT), 1)
    lstrict = (jj < ii).astype(jnp.bfloat16)
    prefix = jnp.dot(lstrict, ohb, preferred_element_type=jnp.float32)
    rank = jnp.sum(prefix * ohf, axis=1, keepdims=True)
    base = jnp.sum(acc_ref[...] * ohf, axis=1, keepdims=True)
    pn_out[...] = base + rank
    acc_ref[...] += jnp.sum(ohf, axis=0, keepdims=True)
    total_out[...] = acc_ref[...]


def _tc_sortb_body(pn_ref, total_ref, rel_ref, pos_out, cumend_out, bounds_out):
    total = total_ref[...]
    ur = (lax.broadcasted_iota(jnp.int32, (R, R), 0)
          <= lax.broadcasted_iota(jnp.int32, (R, R), 1)).astype(jnp.float32)
    cumend = jnp.dot(total, ur, preferred_element_type=jnp.float32)
    goff = cumend - total
    iota_r = lax.broadcasted_iota(jnp.int32, (1, R), 1)
    ohf = (rel_ref[...] == iota_r).astype(jnp.float32)
    posadd = jnp.sum(ohf * goff, axis=1, keepdims=True)
    pos_out[...] = (pn_ref[...] + posadd).astype(jnp.int32)
    cumend_out[...] = cumend
    starts = lax.broadcasted_iota(jnp.int32, (T, 1), 0).astype(jnp.float32) * BT
    rlo = jnp.sum((cumend <= starts).astype(jnp.float32), axis=1, keepdims=True)
    rhi = jnp.sum((cumend <= starts + (BT - 1)).astype(jnp.float32),
                  axis=1, keepdims=True)
    bounds_out[...] = jnp.concatenate([rlo, rhi], axis=1).astype(jnp.int32)


def _tc_score_body(bounds_ref, cumend_ref, zhs_ref, w_ref, out_ref):
    t = pl.program_id(0)
    rlo = bounds_ref[t, 0]
    rhi = bounds_ref[t, 1]
    slot = (lax.broadcasted_iota(jnp.int32, (BT, 1), 0).astype(jnp.float32)
            + jnp.float32(BT) * t.astype(jnp.float32))
    relslot = jnp.sum((cumend_ref[...] <= slot).astype(jnp.float32),
                      axis=1, keepdims=True)
    out_ref[...] = jnp.zeros_like(out_ref)
    zh_bf = zhs_ref[...].astype(jnp.bfloat16)

    def rbody(r, carry):
        m = relslot == r.astype(jnp.float32)
        zm = jnp.where(m, zh_bf, jnp.bfloat16(0))
        out_ref[...] += jnp.dot(zm, w_ref[r], preferred_element_type=jnp.float32)
        return carry

    lax.fori_loop(rlo, rhi + 1, rbody, 0)


def _tc_final_body(tmp_ref, zt_ref, out_ref):
    out_ref[...] = jnp.sum(tmp_ref[...] * zt_ref[...], axis=1, keepdims=True)


def kernel(edge_index, rel_ids, head, tail, x0, W1, b1, W2, b2, rel_W):
    src = edge_index[0]
    dst = edge_index[1]
    degp = _sc_deg(dst, jnp.ones((CH, D), jnp.float32))

    h1p = pl.pallas_call(
        _tc_pre_body,
        out_shape=jax.ShapeDtypeStruct((N, D), jnp.float32),
    )(x0, W1, degp)

    s1p = _sc_msg(src, dst, h1p)

    h2p = pl.pallas_call(
        _tc_mid_body,
        out_shape=jax.ShapeDtypeStruct((N, D), jnp.float32),
    )(s1p, degp, b1, W2)

    s2p = _sc_msg(src, dst, h2p)

    z2, relw_bf = pl.pallas_call(
        _tc_fin_body,
        out_shape=(
            jax.ShapeDtypeStruct((N, D), jnp.float32),
            jax.ShapeDtypeStruct((R, D, D), jnp.bfloat16),
        ),
    )(s2p, degp, b2, rel_W)

    rel2 = rel_ids.reshape(B, 1)

    pn, total = pl.pallas_call(
        _tc_sorta_body,
        grid=(T,),
        in_specs=[pl.BlockSpec((BT, 1), lambda t: (t, 0))],
        out_specs=(
            pl.BlockSpec((BT, 1), lambda t: (t, 0)),
            pl.BlockSpec((1, R), lambda t: (0, 0)),
        ),
        out_shape=(
            jax.ShapeDtypeStruct((B, 1), jnp.float32),
            jax.ShapeDtypeStruct((1, R), jnp.float32),
        ),
        scratch_shapes=[pltpu.VMEM((1, R), jnp.float32)],
    )(rel2)

    pos, cumend, bounds = pl.pallas_call(
        _tc_sortb_body,
        out_shape=(
            jax.ShapeDtypeStruct((B, 1), jnp.int32),
            jax.ShapeDtypeStruct((1, R), jnp.float32),
            jax.ShapeDtypeStruct((T, 2), jnp.int32),
        ),
    )(pn, total, rel2)

    zhs, zt = _sc_gather(z2, head, tail, pos.reshape(B))

    tmps = pl.pallas_call(
        _tc_score_body,
        grid=(T,),
        in_specs=[
            pl.BlockSpec(memory_space=pltpu.SMEM),
            pl.BlockSpec((1, R), lambda t: (0, 0)),
            pl.BlockSpec((BT, D), lambda t: (t, 0)),
            pl.BlockSpec((R, D, D), lambda t: (0, 0, 0)),
        ],
        out_specs=pl.BlockSpec((BT, D), lambda t: (t, 0)),
        out_shape=jax.ShapeDtypeStruct((B, D), jnp.float32),
    )(bounds, cumend, zhs, relw_bf)

    tmp = _sc_unsort(tmps, pos.reshape(B))

    scores = pl.pallas_call(
        _tc_final_body,
        out_shape=jax.ShapeDtypeStruct((B, 1), jnp.float32),
    )(tmp, zt)

    return scores.reshape(B)
